# baseline (speedup 1.0000x reference)
import functools

import jax
import jax.numpy as jnp
from jax import lax
from jax.experimental import pallas as pl
from jax.experimental.pallas import tpu as pltpu
from jax.experimental.pallas import tpu_sc as plsc

N = 10000
D = 128
E = 320000
NC = 2
NS = 16
NW = NC * NS
CH = 80
NSUB = 128
EPAD = NW * NSUB * CH
BQ = 4
NITER = NSUB // BQ
NPAD = 10240
ZR = NPAD // NS

_mesh = plsc.VectorSubcoreMesh(core_axis_name="c", subcore_axis_name="s")


@functools.partial(
    pl.kernel,
    out_type=jax.ShapeDtypeStruct((NC, NPAD), jnp.float32),
    mesh=_mesh,
    scratch_types=[
        pltpu.VMEM((NSUB, 2, CH), jnp.int32),
        pltpu.VMEM((CH,), jnp.float32),
        pltpu.VMEM((ZR,), jnp.float32),
        pltpu.VMEM_SHARED((NPAD,), jnp.float32),
    ],
)
def _sc_count(eidx_hbm, cnt_hbm, idx_v, ones_v, zed_v, acc_sh):
    c = lax.axis_index("c")
    s = lax.axis_index("s")
    w = c * NS + s
    for i in range(CH // 16):
        ones_v[pl.ds(i * 16, 16)] = jnp.ones((16,), jnp.float32)
    for i in range(ZR // 16):
        zed_v[pl.ds(i * 16, 16)] = jnp.zeros((16,), jnp.float32)
    pltpu.sync_copy(zed_v, acc_sh.at[pl.ds(s * ZR, ZR)])
    pltpu.sync_copy(eidx_hbm.at[w], idx_v)
    plsc.subcore_barrier()

    def body(j, carry):
        pltpu.sync_copy(ones_v, acc_sh.at[idx_v.at[j, 1]], add=True)
        return carry

    lax.fori_loop(0, NSUB, body, 0)
    plsc.subcore_barrier()
    pltpu.sync_copy(acc_sh.at[pl.ds(s * ZR, ZR)], cnt_hbm.at[c, pl.ds(s * ZR, ZR)])


@functools.partial(
    pl.kernel,
    out_type=jax.ShapeDtypeStruct((NC, NPAD, D), jnp.float32),
    mesh=_mesh,
    scratch_types=[
        pltpu.VMEM((2, BQ, 2, CH), jnp.int32),
        pltpu.VMEM((CH, D), jnp.float32),
        pltpu.VMEM((CH, D), jnp.float32),
        pltpu.VMEM((CH, D), jnp.float32),
        pltpu.VMEM((CH, D), jnp.float32),
        pltpu.VMEM_SHARED((NPAD, D), jnp.float32),
        pltpu.SemaphoreType.DMA,
        pltpu.SemaphoreType.DMA,
        pltpu.SemaphoreType.DMA,
        pltpu.SemaphoreType.DMA,
        pltpu.SemaphoreType.DMA,
    ],
)
def _sc_scatter(g_hbm, eidx_hbm, zrow_hbm, out_hbm,
                ring_v, r0, r1, r2, r3, acc_sh, sg0, sg1, ss0, ss1, si):
    c = lax.axis_index("c")
    s = lax.axis_index("s")
    w = c * NS + s
    tile_idx = eidx_hbm.at[w]
    pltpu.sync_copy(zrow_hbm, acc_sh.at[pl.ds(s * ZR, ZR)])
    pltpu.sync_copy(tile_idx.at[pl.ds(0, BQ)], ring_v.at[0])
    plsc.subcore_barrier()

    pltpu.async_copy(g_hbm.at[ring_v.at[0, 0, 0]], r0, sg0)
    pltpu.async_copy(g_hbm.at[ring_v.at[0, 1, 0]], r1, sg1)

    def it(i, carry):
        q = i % 2
        qn = (i + 1) % 2

        def gwait(kpos, buf, sg, slot):
            pltpu.make_async_copy(g_hbm.at[ring_v.at[slot, kpos, 0]], buf, sg).wait()

        def swait(kpos, buf, ss, slot):
            pltpu.make_async_copy(buf, acc_sh.at[ring_v.at[slot, kpos, 1]], ss).wait()

        def sstart(kpos, buf, ss, slot):
            pltpu.async_copy(buf, acc_sh.at[ring_v.at[slot, kpos, 1]], ss, add=True)

        def gstart(kpos, buf, sg, slot):
            pltpu.async_copy(g_hbm.at[ring_v.at[slot, kpos, 0]], buf, sg)

        gwait(0, r0, sg0, q)

        @pl.when(i > 0)
        def _():
            swait(2, r2, ss0, qn)
        sstart(0, r0, ss0, q)
        gstart(2, r2, sg0, q)

        gwait(1, r1, sg1, q)

        @pl.when(i > 0)
        def _():
            swait(3, r3, ss1, qn)
        sstart(1, r1, ss1, q)
        gstart(3, r3, sg1, q)

        @pl.when(i + 1 < NITER)
        def _():
            pltpu.async_copy(tile_idx.at[pl.ds((i + 1) * BQ, BQ)], ring_v.at[qn], si)

        gwait(2, r2, sg0, q)
        swait(0, r0, ss0, q)
        sstart(2, r2, ss0, q)

        @pl.when(i + 1 < NITER)
        def _():
            pltpu.make_async_copy(tile_idx.at[pl.ds((i + 1) * BQ, BQ)],
                                  ring_v.at[qn], si).wait()
            gstart(0, r0, sg0, qn)

        gwait(3, r3, sg1, q)
        swait(1, r1, ss1, q)
        sstart(3, r3, ss1, q)

        @pl.when(i + 1 < NITER)
        def _():
            gstart(1, r1, sg1, qn)

        return carry

    lax.fori_loop(0, NITER, it, 0)
    pltpu.make_async_copy(r2, acc_sh.at[ring_v.at[1, 2, 1]], ss0).wait()
    pltpu.make_async_copy(r3, acc_sh.at[ring_v.at[1, 3, 1]], ss1).wait()
    plsc.subcore_barrier()
    pltpu.sync_copy(acc_sh.at[pl.ds(s * ZR, ZR)], out_hbm.at[c, pl.ds(s * ZR, ZR)])


RB = 2000


def _dis(cnt_ref):
    return lax.rsqrt(cnt_ref[0] + cnt_ref[1] + 1.0)


def _pre_body(x_ref, w_ref, cnt_ref, g_ref):
    g_ref[...] = jnp.dot(x_ref[...], w_ref[...],
                         preferred_element_type=jnp.float32) * _dis(cnt_ref)


def _mid_body(s_ref, g_ref, cnt_ref, w_ref, b_ref, out_ref):
    dis = _dis(cnt_ref)
    p = dis * (s_ref[0] + s_ref[1] + g_ref[...]) + b_ref[...]
    h = jnp.maximum(p, 0.0)
    out_ref[...] = jnp.dot(h, w_ref[...],
                           preferred_element_type=jnp.float32) * dis


def _post_body(s_ref, g_ref, cnt_ref, b_ref, out_ref):
    dis = _dis(cnt_ref)
    out_ref[...] = dis * (s_ref[0] + s_ref[1] + g_ref[...]) + b_ref[...]


_s_spec = pl.BlockSpec((NC, RB, D), lambda r: (0, r, 0))
_row_spec = pl.BlockSpec((RB, D), lambda r: (r, 0))
_w_spec = pl.BlockSpec((D, D), lambda r: (0, 0))
_cnt_spec = pl.BlockSpec((NC, RB, 1), lambda r: (0, r, 0))
_b_spec = pl.BlockSpec((1, D), lambda r: (0, 0))
_out_row = jax.ShapeDtypeStruct((N, D), jnp.float32)

_pre = pl.pallas_call(
    _pre_body,
    grid=(N // RB,),
    in_specs=[_row_spec, _w_spec, _cnt_spec],
    out_specs=_row_spec,
    out_shape=_out_row,
)

_mid = pl.pallas_call(
    _mid_body,
    grid=(N // RB,),
    in_specs=[_s_spec, _row_spec, _cnt_spec, _w_spec, _b_spec],
    out_specs=_row_spec,
    out_shape=_out_row,
)

_post = pl.pallas_call(
    _post_body,
    grid=(N // RB,),
    in_specs=[_s_spec, _row_spec, _cnt_spec, _b_spec],
    out_specs=_row_spec,
    out_shape=_out_row,
)


@jax.jit
def kernel(x, edge_index, W1, b1, W2, b2):
    ei = edge_index.astype(jnp.int32)
    pad_src = jnp.arange(EPAD - E, dtype=jnp.int32) % N
    pad_dst = N + (jnp.arange(EPAD - E, dtype=jnp.int32) % (NPAD - N))
    srcp = jnp.concatenate([ei[0], pad_src]).reshape(NW, NSUB, CH)
    dstp = jnp.concatenate([ei[1], pad_dst]).reshape(NW, NSUB, CH)
    eidx = jnp.stack([srcp, dstp], axis=2)
    zrow = jnp.zeros((ZR, D), jnp.float32)
    b1r = b1.reshape(1, D)
    b2r = b2.reshape(1, D)

    cnt = _sc_count(eidx)
    cnt3 = cnt.reshape(NC, NPAD, 1)
    g1 = _pre(x, W1, cnt3)
    s1 = _sc_scatter(g1, eidx, zrow)
    g2 = _mid(s1, g1, cnt3, W2, b1r)
    s2 = _sc_scatter(g2, eidx, zrow)
    return _post(s2, g2, cnt3, b2r)


# speedup vs baseline: 1.0785x; 1.0785x over previous
import functools

import jax
import jax.numpy as jnp
from jax import lax
from jax.experimental import pallas as pl
from jax.experimental.pallas import tpu as pltpu
from jax.experimental.pallas import tpu_sc as plsc

N = 10000
D = 128
E = 320000
NC = 2
NS = 16
NW = NC * NS
CH = 128
NSUB = 80
EPAD = NW * NSUB * CH
BQ = 20
NBLK = NSUB // BQ
NPAD = 10240
ZR = NPAD // NS

_mesh = plsc.VectorSubcoreMesh(core_axis_name="c", subcore_axis_name="s")


@functools.partial(
    pl.kernel,
    out_type=jax.ShapeDtypeStruct((NC, NPAD), jnp.float32),
    mesh=_mesh,
    scratch_types=[
        pltpu.VMEM((NSUB, 2, CH), jnp.int32),
        pltpu.VMEM((CH,), jnp.float32),
        pltpu.VMEM((ZR,), jnp.float32),
        pltpu.VMEM_SHARED((NPAD,), jnp.float32),
    ],
)
def _sc_count(eidx_hbm, cnt_hbm, idx_v, ones_v, zed_v, acc_sh):
    c = lax.axis_index("c")
    s = lax.axis_index("s")
    w = c * NS + s
    for i in range(CH // 16):
        ones_v[pl.ds(i * 16, 16)] = jnp.ones((16,), jnp.float32)
    for i in range(ZR // 16):
        zed_v[pl.ds(i * 16, 16)] = jnp.zeros((16,), jnp.float32)
    pltpu.sync_copy(zed_v, acc_sh.at[pl.ds(s * ZR, ZR)])
    pltpu.sync_copy(eidx_hbm.at[w], idx_v)
    plsc.subcore_barrier()

    def body(j, carry):
        pltpu.sync_copy(ones_v, acc_sh.at[idx_v.at[j, 1]], add=True)
        return carry

    lax.fori_loop(0, NSUB, body, 0)
    plsc.subcore_barrier()
    pltpu.sync_copy(acc_sh.at[pl.ds(s * ZR, ZR)], cnt_hbm.at[c, pl.ds(s * ZR, ZR)])


@functools.partial(
    pl.kernel,
    out_type=jax.ShapeDtypeStruct((NC, NPAD, D), jnp.float32),
    mesh=_mesh,
    scratch_types=[
        pltpu.VMEM((2, BQ, 2, CH), jnp.int32),
        pltpu.VMEM((CH, D), jnp.float32),
        pltpu.VMEM((CH, D), jnp.float32),
        pltpu.VMEM_SHARED((NPAD, D), jnp.float32),
        pltpu.SemaphoreType.DMA,
        pltpu.SemaphoreType.DMA,
    ],
)
def _sc_scatter(g_hbm, eidx_hbm, zrow_hbm, out_hbm,
                ring_v, rows0_v, rows1_v, acc_sh, sem0, sem1):
    c = lax.axis_index("c")
    s = lax.axis_index("s")
    w = c * NS + s
    tile_idx = eidx_hbm.at[w]
    pltpu.sync_copy(zrow_hbm, acc_sh.at[pl.ds(s * ZR, ZR)])
    pltpu.sync_copy(tile_idx.at[pl.ds(0, BQ)], ring_v.at[0])
    plsc.subcore_barrier()

    pltpu.async_copy(g_hbm.at[ring_v.at[0, 0, 0]], rows0_v, sem0)

    def blk(b, carry):
        q = b % 2
        qn = (b + 1) % 2
        cur = ring_v.at[q]

        @pl.when(b + 1 < NBLK)
        def _():
            pltpu.sync_copy(tile_idx.at[pl.ds((b + 1) * BQ, BQ)], ring_v.at[qn])

        def pair(k2, carry2):
            k = 2 * k2
            pltpu.async_copy(g_hbm.at[cur.at[k + 1, 0]], rows1_v, sem1)
            pltpu.make_async_copy(g_hbm.at[cur.at[k, 0]], rows0_v, sem0).wait()
            pltpu.sync_copy(rows0_v, acc_sh.at[cur.at[k, 1]], add=True)

            @pl.when(k2 + 1 < BQ // 2)
            def _():
                pltpu.async_copy(g_hbm.at[cur.at[k + 2, 0]], rows0_v, sem0)

            pltpu.make_async_copy(g_hbm.at[cur.at[k + 1, 0]], rows1_v, sem1).wait()
            pltpu.sync_copy(rows1_v, acc_sh.at[cur.at[k + 1, 1]], add=True)
            return carry2

        lax.fori_loop(0, BQ // 2, pair, 0)

        @pl.when(b + 1 < NBLK)
        def _():
            pltpu.async_copy(g_hbm.at[ring_v.at[qn, 0, 0]], rows0_v, sem0)

        return carry

    lax.fori_loop(0, NBLK, blk, 0)
    plsc.subcore_barrier()
    pltpu.sync_copy(acc_sh.at[pl.ds(s * ZR, ZR)], out_hbm.at[c, pl.ds(s * ZR, ZR)])


RB = 2000


def _dis(cnt_ref):
    return lax.rsqrt(cnt_ref[0] + cnt_ref[1] + 1.0)


def _pre_body(x_ref, w_ref, cnt_ref, g_ref):
    g_ref[...] = jnp.dot(x_ref[...], w_ref[...],
                         preferred_element_type=jnp.float32) * _dis(cnt_ref)


def _mid_body(s_ref, g_ref, cnt_ref, w_ref, b_ref, out_ref):
    dis = _dis(cnt_ref)
    p = dis * (s_ref[0] + s_ref[1] + g_ref[...]) + b_ref[...]
    h = jnp.maximum(p, 0.0)
    out_ref[...] = jnp.dot(h, w_ref[...],
                           preferred_element_type=jnp.float32) * dis


def _post_body(s_ref, g_ref, cnt_ref, b_ref, out_ref):
    dis = _dis(cnt_ref)
    out_ref[...] = dis * (s_ref[0] + s_ref[1] + g_ref[...]) + b_ref[...]


_s_spec = pl.BlockSpec((NC, RB, D), lambda r: (0, r, 0))
_row_spec = pl.BlockSpec((RB, D), lambda r: (r, 0))
_w_spec = pl.BlockSpec((D, D), lambda r: (0, 0))
_cnt_spec = pl.BlockSpec((NC, RB, 1), lambda r: (0, r, 0))
_b_spec = pl.BlockSpec((1, D), lambda r: (0, 0))
_out_row = jax.ShapeDtypeStruct((N, D), jnp.float32)

_pre = pl.pallas_call(
    _pre_body,
    grid=(N // RB,),
    in_specs=[_row_spec, _w_spec, _cnt_spec],
    out_specs=_row_spec,
    out_shape=_out_row,
)

_mid = pl.pallas_call(
    _mid_body,
    grid=(N // RB,),
    in_specs=[_s_spec, _row_spec, _cnt_spec, _w_spec, _b_spec],
    out_specs=_row_spec,
    out_shape=_out_row,
)

_post = pl.pallas_call(
    _post_body,
    grid=(N // RB,),
    in_specs=[_s_spec, _row_spec, _cnt_spec, _b_spec],
    out_specs=_row_spec,
    out_shape=_out_row,
)


@jax.jit
def kernel(x, edge_index, W1, b1, W2, b2):
    ei = edge_index.astype(jnp.int32)
    pad_src = jnp.arange(EPAD - E, dtype=jnp.int32) % N
    pad_dst = N + (jnp.arange(EPAD - E, dtype=jnp.int32) % (NPAD - N))
    srcp = jnp.concatenate([ei[0], pad_src]).reshape(NW, NSUB, CH)
    dstp = jnp.concatenate([ei[1], pad_dst]).reshape(NW, NSUB, CH)
    eidx = jnp.stack([srcp, dstp], axis=2)
    zrow = jnp.zeros((ZR, D), jnp.float32)
    b1r = b1.reshape(1, D)
    b2r = b2.reshape(1, D)

    cnt = _sc_count(eidx)
    cnt3 = cnt.reshape(NC, NPAD, 1)
    g1 = _pre(x, W1, cnt3)
    s1 = _sc_scatter(g1, eidx, zrow)
    g2 = _mid(s1, g1, cnt3, W2, b1r)
    s2 = _sc_scatter(g2, eidx, zrow)
    return _post(s2, g2, cnt3, b2r)
